# baseline (speedup 1.0000x reference)
import functools

import jax
import jax.numpy as jnp
from jax.experimental import pallas as pl
from jax.experimental.pallas import tpu as pltpu


TOPK = 8
NUM_EXPERTS = 64
CHUNK = 256
NBUF = 8
LOOKAHEAD = 6


def _topk_block(scores):
    st = scores.T
    m = jnp.max(st, axis=0, keepdims=True)
    e = jnp.exp(st - m)
    s = jnp.sum(e, axis=0, keepdims=True)
    p = e / s
    iota = jax.lax.broadcasted_iota(jnp.int32, p.shape, 0)
    vals = []
    idxs = []
    for _ in range(TOPK):
        mk = jnp.max(p, axis=0, keepdims=True)
        ik = jnp.min(jnp.where(p == mk, iota, NUM_EXPERTS), axis=0,
                     keepdims=True)
        vals.append(mk)
        idxs.append(ik)
        p = jnp.where(iota == ik, -1.0, p)
    return (jnp.concatenate(vals, axis=0).T,
            jnp.concatenate(idxs, axis=0).T)


def _router_kernel(x_ref, wt_ref, w_out_ref, i_out_ref, buf, sems):
    i = pl.program_id(0)
    n = pl.num_programs(0)

    def start(j):
        slot = jax.lax.rem(j, NBUF)
        pltpu.make_async_copy(
            x_ref.at[pl.ds(j * CHUNK, CHUNK), :],
            buf.at[slot],
            sems.at[slot],
        ).start()

    @pl.when(i == 0)
    def _prologue():
        for j in range(LOOKAHEAD):
            start(jnp.int32(j))

    @pl.when(i + LOOKAHEAD < n)
    def _prefetch():
        start(i + LOOKAHEAD)

    slot = jax.lax.rem(i, NBUF)
    pltpu.make_async_copy(
        x_ref.at[pl.ds(i * CHUNK, CHUNK), :],
        buf.at[slot],
        sems.at[slot],
    ).wait()

    x = buf[slot]
    scores = jnp.dot(x, wt_ref[...], preferred_element_type=jnp.float32)
    w_vals, w_idxs = _topk_block(scores)
    w_out_ref[...] = w_vals
    i_out_ref[...] = w_idxs


@functools.partial(jax.jit, static_argnames=())
def kernel(x, weight):
    n_rows = x.shape[0]
    dim = x.shape[1]
    wt = weight.T
    grid = (n_rows // CHUNK,)
    weights_out, indices_out = pl.pallas_call(
        _router_kernel,
        grid=grid,
        in_specs=[
            pl.BlockSpec(memory_space=pl.ANY),
            pl.BlockSpec((dim, NUM_EXPERTS), lambda i: (0, 0)),
        ],
        out_specs=[
            pl.BlockSpec((CHUNK, TOPK), lambda i: (i, 0)),
            pl.BlockSpec((CHUNK, TOPK), lambda i: (i, 0)),
        ],
        out_shape=[
            jax.ShapeDtypeStruct((n_rows, TOPK), jnp.float32),
            jax.ShapeDtypeStruct((n_rows, TOPK), jnp.int32),
        ],
        scratch_shapes=[
            pltpu.VMEM((NBUF, CHUNK, dim), jnp.float32),
            pltpu.SemaphoreType.DMA((NBUF,)),
        ],
    )(x, wt)
    return weights_out, indices_out


# speedup vs baseline: 1.1785x; 1.1785x over previous
import functools

import jax
import jax.numpy as jnp
from jax.experimental import pallas as pl
from jax.experimental.pallas import tpu as pltpu


TOPK = 8
NUM_EXPERTS = 64
CHUNK = 1024
NBUF = 4
LOOKAHEAD = 3


def _topk_block(scores):
    st = scores.T
    m = jnp.max(st, axis=0, keepdims=True)
    e = jnp.exp(st - m)
    s = jnp.sum(e, axis=0, keepdims=True)
    p = e / s
    iota = jax.lax.broadcasted_iota(jnp.int32, p.shape, 0)
    vals = []
    idxs = []
    for _ in range(TOPK):
        mk = jnp.max(p, axis=0, keepdims=True)
        ik = jnp.min(jnp.where(p == mk, iota, NUM_EXPERTS), axis=0,
                     keepdims=True)
        vals.append(mk)
        idxs.append(ik)
        p = jnp.where(iota == ik, -1.0, p)
    return (jnp.concatenate(vals, axis=0).T,
            jnp.concatenate(idxs, axis=0).T)


def _router_kernel(x_ref, wt_ref, w_out_ref, i_out_ref, buf, sems):
    i = pl.program_id(0)
    n = pl.num_programs(0)

    def start(j):
        slot = jax.lax.rem(j, NBUF)
        pltpu.make_async_copy(
            x_ref.at[pl.ds(j * CHUNK, CHUNK), :],
            buf.at[slot],
            sems.at[slot],
        ).start()

    @pl.when(i == 0)
    def _prologue():
        for j in range(LOOKAHEAD):
            start(jnp.int32(j))

    @pl.when(i + LOOKAHEAD < n)
    def _prefetch():
        start(i + LOOKAHEAD)

    slot = jax.lax.rem(i, NBUF)
    pltpu.make_async_copy(
        x_ref.at[pl.ds(i * CHUNK, CHUNK), :],
        buf.at[slot],
        sems.at[slot],
    ).wait()

    x = buf[slot]
    scores = jnp.dot(x, wt_ref[...], preferred_element_type=jnp.float32)
    w_vals, w_idxs = _topk_block(scores)
    w_out_ref[...] = w_vals
    i_out_ref[...] = w_idxs


@functools.partial(jax.jit, static_argnames=())
def kernel(x, weight):
    n_rows = x.shape[0]
    dim = x.shape[1]
    wt = weight.T
    grid = (n_rows // CHUNK,)
    weights_out, indices_out = pl.pallas_call(
        _router_kernel,
        grid=grid,
        in_specs=[
            pl.BlockSpec(memory_space=pl.ANY),
            pl.BlockSpec((dim, NUM_EXPERTS), lambda i: (0, 0)),
        ],
        out_specs=[
            pl.BlockSpec((CHUNK, TOPK), lambda i: (i, 0)),
            pl.BlockSpec((CHUNK, TOPK), lambda i: (i, 0)),
        ],
        out_shape=[
            jax.ShapeDtypeStruct((n_rows, TOPK), jnp.float32),
            jax.ShapeDtypeStruct((n_rows, TOPK), jnp.int32),
        ],
        scratch_shapes=[
            pltpu.VMEM((NBUF, CHUNK, dim), jnp.float32),
            pltpu.SemaphoreType.DMA((NBUF,)),
        ],
    )(x, wt)
    return weights_out, indices_out


# speedup vs baseline: 1.2865x; 1.0917x over previous
import functools

import jax
import jax.numpy as jnp
from jax.experimental import pallas as pl
from jax.experimental.pallas import tpu as pltpu


TOPK = 8
NUM_EXPERTS = 64
CHUNK = 512
NBUF = 4
LOOKAHEAD = 3


def _topk_block(scores):
    st = scores.T
    m = jnp.max(st, axis=0, keepdims=True)
    e = jnp.exp(st - m)
    s = jnp.sum(e, axis=0, keepdims=True)
    p = e / s
    iota = jax.lax.broadcasted_iota(jnp.int32, p.shape, 0)
    vals = []
    idxs = []
    for _ in range(TOPK):
        mk = jnp.max(p, axis=0, keepdims=True)
        ik = jnp.min(jnp.where(p == mk, iota, NUM_EXPERTS), axis=0,
                     keepdims=True)
        vals.append(mk)
        idxs.append(ik)
        p = jnp.where(iota == ik, -1.0, p)
    return (jnp.concatenate(vals, axis=0).T,
            jnp.concatenate(idxs, axis=0).T)


def _router_kernel(x_ref, w_ref, w_out_ref, i_out_ref, buf, sems, wt_scr):
    i = pl.program_id(0)
    n = pl.num_programs(0)

    def start(j):
        slot = jax.lax.rem(j, NBUF)
        pltpu.make_async_copy(
            x_ref.at[pl.ds(j * CHUNK, CHUNK), :],
            buf.at[slot],
            sems.at[slot],
        ).start()

    @pl.when(i == 0)
    def _prologue():
        for j in range(LOOKAHEAD):
            start(jnp.int32(j))
        wt_scr[...] = w_ref[...].T

    @pl.when(i + LOOKAHEAD < n)
    def _prefetch():
        start(i + LOOKAHEAD)

    slot = jax.lax.rem(i, NBUF)
    pltpu.make_async_copy(
        x_ref.at[pl.ds(i * CHUNK, CHUNK), :],
        buf.at[slot],
        sems.at[slot],
    ).wait()

    x = buf[slot]
    scores = jnp.dot(x, wt_scr[...], preferred_element_type=jnp.float32)
    w_vals, w_idxs = _topk_block(scores)
    w_out_ref[...] = w_vals
    i_out_ref[...] = w_idxs


@functools.partial(jax.jit, static_argnames=())
def kernel(x, weight):
    n_rows = x.shape[0]
    dim = x.shape[1]
    grid = (n_rows // CHUNK,)
    weights_out, indices_out = pl.pallas_call(
        _router_kernel,
        grid=grid,
        in_specs=[
            pl.BlockSpec(memory_space=pl.ANY),
            pl.BlockSpec((NUM_EXPERTS, dim), lambda i: (0, 0)),
        ],
        out_specs=[
            pl.BlockSpec((CHUNK, TOPK), lambda i: (i, 0)),
            pl.BlockSpec((CHUNK, TOPK), lambda i: (i, 0)),
        ],
        out_shape=[
            jax.ShapeDtypeStruct((n_rows, TOPK), jnp.float32),
            jax.ShapeDtypeStruct((n_rows, TOPK), jnp.int32),
        ],
        scratch_shapes=[
            pltpu.VMEM((NBUF, CHUNK, dim), jnp.float32),
            pltpu.SemaphoreType.DMA((NBUF,)),
            pltpu.VMEM((dim, NUM_EXPERTS), jnp.float32),
        ],
    )(x, weight)
    return weights_out, indices_out


# speedup vs baseline: 1.2872x; 1.0006x over previous
import functools

import jax
import jax.numpy as jnp
from jax.experimental import pallas as pl
from jax.experimental.pallas import tpu as pltpu


TOPK = 8
NUM_EXPERTS = 64
CHUNK = 512
NBUF = 6
LOOKAHEAD = 5


def _topk_block(scores):
    st = scores.T
    m = jnp.max(st, axis=0, keepdims=True)
    e = jnp.exp(st - m)
    s = jnp.sum(e, axis=0, keepdims=True)
    p = e / s
    iota = jax.lax.broadcasted_iota(jnp.int32, p.shape, 0)
    vals = []
    idxs = []
    for _ in range(TOPK):
        mk = jnp.max(p, axis=0, keepdims=True)
        ik = jnp.min(jnp.where(p == mk, iota, NUM_EXPERTS), axis=0,
                     keepdims=True)
        vals.append(mk)
        idxs.append(ik)
        p = jnp.where(iota == ik, -1.0, p)
    return (jnp.concatenate(vals, axis=0).T,
            jnp.concatenate(idxs, axis=0).T)


def _router_kernel(x_ref, w_ref, w_out_ref, i_out_ref, buf, sems, wt_scr):
    i = pl.program_id(0)
    n = pl.num_programs(0)

    def start(j):
        slot = jax.lax.rem(j, NBUF)
        pltpu.make_async_copy(
            x_ref.at[pl.ds(j * CHUNK, CHUNK), :],
            buf.at[slot],
            sems.at[slot],
        ).start()

    @pl.when(i == 0)
    def _prologue():
        for j in range(LOOKAHEAD):
            start(jnp.int32(j))
        wt_scr[...] = w_ref[...].T

    @pl.when(i + LOOKAHEAD < n)
    def _prefetch():
        start(i + LOOKAHEAD)

    slot = jax.lax.rem(i, NBUF)
    pltpu.make_async_copy(
        x_ref.at[pl.ds(i * CHUNK, CHUNK), :],
        buf.at[slot],
        sems.at[slot],
    ).wait()

    x = buf[slot]
    scores = jnp.dot(x, wt_scr[...], preferred_element_type=jnp.float32)
    w_vals, w_idxs = _topk_block(scores)
    w_out_ref[...] = w_vals
    i_out_ref[...] = w_idxs


@functools.partial(jax.jit, static_argnames=())
def kernel(x, weight):
    n_rows = x.shape[0]
    dim = x.shape[1]
    grid = (n_rows // CHUNK,)
    weights_out, indices_out = pl.pallas_call(
        _router_kernel,
        grid=grid,
        in_specs=[
            pl.BlockSpec(memory_space=pl.ANY),
            pl.BlockSpec((NUM_EXPERTS, dim), lambda i: (0, 0)),
        ],
        out_specs=[
            pl.BlockSpec((CHUNK, TOPK), lambda i: (i, 0)),
            pl.BlockSpec((CHUNK, TOPK), lambda i: (i, 0)),
        ],
        out_shape=[
            jax.ShapeDtypeStruct((n_rows, TOPK), jnp.float32),
            jax.ShapeDtypeStruct((n_rows, TOPK), jnp.int32),
        ],
        scratch_shapes=[
            pltpu.VMEM((NBUF, CHUNK, dim), jnp.float32),
            pltpu.SemaphoreType.DMA((NBUF,)),
            pltpu.VMEM((dim, NUM_EXPERTS), jnp.float32),
        ],
    )(x, weight)
    return weights_out, indices_out
